# baseline (device time: 79890 ns/iter reference)
import jax
import jax.numpy as jnp
from jax import lax
from jax.experimental import pallas as pl
from jax.experimental.pallas import tpu as pltpu

N_DEV = 16


def kernel(x, w_mat):
    m_loc, k = x.shape
    _, n = w_mat.shape
    n_loc = n // N_DEV
    m = m_loc * N_DEV

    my = lax.axis_index("i")
    perm = (my + 1 + jnp.arange(N_DEV, dtype=jnp.int32)) % N_DEV

    def body(perm_ref, x_ref, w_ref, out_ref,
             asm, send_buf, scal_buf, amax_ref,
             send_sems, recv_sems, ssend_sems, srecv_sems):
        t = pl.program_id(0)
        me = lax.axis_index("i")
        d = perm_ref[t]

        blk = jnp.dot(x_ref[...], w_ref[...],
                      preferred_element_type=jnp.float32)
        bb = blk.astype(jnp.bfloat16)
        bmax = jnp.max(jnp.abs(bb.astype(jnp.float32)))

        @pl.when(t == 0)
        def _():
            amax_ref[0] = bmax

        @pl.when(t != 0)
        def _():
            amax_ref[0] = jnp.maximum(amax_ref[0], bmax)

        @pl.when(t < N_DEV - 1)
        def _():
            send_buf[d] = bb
            rdma = pltpu.make_async_remote_copy(
                src_ref=send_buf.at[d],
                dst_ref=asm.at[me],
                send_sem=send_sems.at[d],
                recv_sem=recv_sems.at[me],
                device_id=(d,),
                device_id_type=pl.DeviceIdType.MESH,
            )
            rdma.start()

        @pl.when(t == N_DEV - 1)
        def _():
            asm[me] = bb

            amax_l = amax_ref[0]
            scal_buf[me] = jnp.full((8, 128), amax_l, jnp.float32)
            for s in range(1, N_DEV):
                dd = (me + s) % N_DEV
                sc = pltpu.make_async_remote_copy(
                    src_ref=scal_buf.at[me],
                    dst_ref=scal_buf.at[me],
                    send_sem=ssend_sems.at[dd],
                    recv_sem=srecv_sems.at[me],
                    device_id=(dd,),
                    device_id_type=pl.DeviceIdType.MESH,
                )
                sc.start()

            for s in range(1, N_DEV):
                src = (me + s) % N_DEV
                rcv = pltpu.make_async_remote_copy(
                    src_ref=asm.at[src],
                    dst_ref=asm.at[src],
                    send_sem=send_sems.at[src],
                    recv_sem=recv_sems.at[src],
                    device_id=(src,),
                    device_id_type=pl.DeviceIdType.MESH,
                )
                rcv.wait_recv()
                srcv = pltpu.make_async_remote_copy(
                    src_ref=scal_buf.at[src],
                    dst_ref=scal_buf.at[src],
                    send_sem=ssend_sems.at[src],
                    recv_sem=srecv_sems.at[src],
                    device_id=(src,),
                    device_id_type=pl.DeviceIdType.MESH,
                )
                srcv.wait_recv()

            for s in range(1, N_DEV):
                dd = (me + s) % N_DEV
                snd = pltpu.make_async_remote_copy(
                    src_ref=send_buf.at[dd],
                    dst_ref=asm.at[me],
                    send_sem=send_sems.at[dd],
                    recv_sem=recv_sems.at[me],
                    device_id=(dd,),
                    device_id_type=pl.DeviceIdType.MESH,
                )
                snd.wait_send()
                ssnd = pltpu.make_async_remote_copy(
                    src_ref=scal_buf.at[me],
                    dst_ref=scal_buf.at[me],
                    send_sem=ssend_sems.at[dd],
                    recv_sem=srecv_sems.at[me],
                    device_id=(dd,),
                    device_id_type=pl.DeviceIdType.MESH,
                )
                ssnd.wait_send()

            amax_g = jnp.max(scal_buf[...])
            scale = amax_g / 448.0
            for i in range(N_DEV):
                y = asm[i].astype(jnp.float32)
                q = jnp.clip(y / scale, -448.0, 448.0)
                q = q.astype(jnp.float8_e4m3fn).astype(jnp.float32) * scale
                out_ref[pl.ds(i * m_loc, m_loc), :] = q

    grid_spec = pltpu.PrefetchScalarGridSpec(
        num_scalar_prefetch=1,
        grid=(N_DEV,),
        in_specs=[
            pl.BlockSpec((m_loc, k), lambda t, p: (0, 0)),
            pl.BlockSpec((k, n_loc), lambda t, p: (0, p[t])),
        ],
        out_specs=pl.BlockSpec((m, n_loc), lambda t, p: (0, 0)),
        scratch_shapes=[
            pltpu.VMEM((N_DEV, m_loc, n_loc), jnp.bfloat16),
            pltpu.VMEM((N_DEV, m_loc, n_loc), jnp.bfloat16),
            pltpu.VMEM((N_DEV, 8, 128), jnp.float32),
            pltpu.SMEM((1,), jnp.float32),
            pltpu.SemaphoreType.DMA((N_DEV,)),
            pltpu.SemaphoreType.DMA((N_DEV,)),
            pltpu.SemaphoreType.DMA((N_DEV,)),
            pltpu.SemaphoreType.DMA((N_DEV,)),
        ],
    )
    return pl.pallas_call(
        body,
        grid_spec=grid_spec,
        out_shape=jax.ShapeDtypeStruct((m, n_loc), jnp.float32),
        compiler_params=pltpu.CompilerParams(
            dimension_semantics=("arbitrary",),
        ),
    )(perm, x, w_mat)
